# baseline (speedup 1.0000x reference)
import jax
import jax.numpy as jnp
from jax import lax
from jax.experimental import pallas as pl
from jax.experimental.pallas import tpu as pltpu

NUM_USERS = 1000000
NUM_ITEMS = 100000
EMB = 32
BATCH = 1024

_BLOCK_N = 2560
_NJ = (NUM_ITEMS + _BLOCK_N - 1) // _BLOCK_N
_G = BATCH // 128
_SLAB_K = 128 * 128


def _body(idx_ref, ueT_hbm, itT_hbm, idxv_ref, out_ref,
          uvT_ref, itT_ref, slabs_ref, sem_g, sem_i):
    j = pl.program_id(0)

    @pl.when(j == 0)
    def _stage():
        pltpu.make_async_copy(itT_hbm, itT_ref, sem_i).start()

        for g0 in range(_G):
            def issue(b, carry, g0=g0):
                r = idx_ref[b]
                slab = jnp.minimum((r >> 7) << 7, NUM_USERS - 128)
                slab = pl.multiple_of(slab, 128)
                dst = pl.multiple_of(b * 128, 128)
                pltpu.make_async_copy(
                    ueT_hbm.at[:, pl.ds(slab, 128)],
                    slabs_ref.at[:, pl.ds(dst, 128)],
                    sem_g.at[g0],
                ).start()
                return carry

            lax.fori_loop(g0 * 128, (g0 + 1) * 128, issue, 0)

        for g in range(_G):
            iv = idxv_ref[pl.ds(g, 1), :]
            slab_start = jnp.minimum((iv >> 7) << 7, NUM_USERS - 128)
            lane = iv - slab_start
            col = jax.lax.broadcasted_iota(jnp.int32, (1, 128), 1)
            target = col * 128 + lane
            rows = jax.lax.broadcasted_iota(jnp.int32, (_SLAB_K, 128), 0)
            onehot = jnp.where(
                rows == jnp.broadcast_to(target, (_SLAB_K, 128)),
                jnp.float32(1),
                jnp.float32(0),
            ).astype(jnp.bfloat16)
            grp = slabs_ref.at[:, pl.ds(g * _SLAB_K, _SLAB_K)]
            pltpu.make_async_copy(grp, grp, sem_g.at[g]).wait()
            slab_val = slabs_ref[:, pl.ds(g * _SLAB_K, _SLAB_K)]
            uvg = lax.dot_general(
                slab_val.astype(jnp.bfloat16),
                onehot,
                dimension_numbers=(((1,), (0,)), ((), ())),
                preferred_element_type=jnp.float32,
            )
            uvT_ref[:, pl.ds(g * 128, 128)] = uvg

        pltpu.make_async_copy(itT_hbm, itT_ref, sem_i).wait()

    out_ref[...] = lax.dot_general(
        itT_ref[:, pl.ds(j * _BLOCK_N, _BLOCK_N)],
        uvT_ref[...],
        dimension_numbers=(((0,), (0,)), ((), ())),
        preferred_element_type=jnp.float32,
    )


_grid_spec = pltpu.PrefetchScalarGridSpec(
    num_scalar_prefetch=1,
    grid=(_NJ,),
    in_specs=[
        pl.BlockSpec(memory_space=pltpu.MemorySpace.HBM),
        pl.BlockSpec(memory_space=pltpu.MemorySpace.HBM),
        pl.BlockSpec((_G, 128), lambda j, idx: (0, 0)),
    ],
    out_specs=pl.BlockSpec((_BLOCK_N, BATCH), lambda j, idx: (j, 0)),
    scratch_shapes=[
        pltpu.VMEM((EMB, BATCH), jnp.float32),
        pltpu.VMEM((EMB, NUM_ITEMS), jnp.float32),
        pltpu.VMEM((EMB, 128 * BATCH), jnp.float32),
        pltpu.SemaphoreType.DMA((_G,)),
        pltpu.SemaphoreType.DMA,
    ],
)

_fused = pl.pallas_call(
    _body,
    grid_spec=_grid_spec,
    out_shape=jax.ShapeDtypeStruct((NUM_ITEMS, BATCH), jnp.float32),
    compiler_params=pltpu.CompilerParams(
        dimension_semantics=("arbitrary",),
    ),
)


def kernel(u_idx_tensor, u_emb, i_emb):
    idx = u_idx_tensor.astype(jnp.int32)
    idx2 = idx.reshape(_G, 128)
    return _fused(idx, u_emb.T, i_emb.T, idx2).T


# speedup vs baseline: 1.0096x; 1.0096x over previous
import jax
import jax.numpy as jnp
from jax import lax
from jax.experimental import pallas as pl
from jax.experimental.pallas import tpu as pltpu

NUM_USERS = 1000000
NUM_ITEMS = 100000
EMB = 32
BATCH = 1024

_BLOCK_N = 2048
_NJ = (NUM_ITEMS + _BLOCK_N - 1) // _BLOCK_N
_G = BATCH // 128
_SLAB_K = 128 * 128


def _body(idx_ref, ueT_hbm, itT_hbm, idxv_ref, out_ref,
          uvT_ref, itT_ref, slabs_ref, sem_g, sem_i):
    j = pl.program_id(0)

    @pl.when(j == 0)
    def _stage():
        pltpu.make_async_copy(itT_hbm, itT_ref, sem_i).start()

        for g0 in range(_G):
            def issue(b, carry, g0=g0):
                r = idx_ref[b]
                slab = jnp.minimum((r >> 7) << 7, NUM_USERS - 128)
                slab = pl.multiple_of(slab, 128)
                dst = pl.multiple_of(b * 128, 128)
                pltpu.make_async_copy(
                    ueT_hbm.at[:, pl.ds(slab, 128)],
                    slabs_ref.at[:, pl.ds(dst, 128)],
                    sem_g.at[g0],
                ).start()
                return carry

            lax.fori_loop(g0 * 128, (g0 + 1) * 128, issue, 0)

        for g in range(_G):
            iv = idxv_ref[pl.ds(g, 1), :]
            slab_start = jnp.minimum((iv >> 7) << 7, NUM_USERS - 128)
            lane = iv - slab_start
            col = jax.lax.broadcasted_iota(jnp.int32, (1, 128), 1)
            target = col * 128 + lane
            rows = jax.lax.broadcasted_iota(jnp.int32, (_SLAB_K, 128), 0)
            onehot = (
                rows == jnp.broadcast_to(target, (_SLAB_K, 128))
            ).astype(jnp.bfloat16)
            grp = slabs_ref.at[:, pl.ds(g * _SLAB_K, _SLAB_K)]
            pltpu.make_async_copy(grp, grp, sem_g.at[g]).wait()
            slab_val = slabs_ref[:, pl.ds(g * _SLAB_K, _SLAB_K)]
            uvg = lax.dot_general(
                slab_val.astype(jnp.bfloat16),
                onehot,
                dimension_numbers=(((1,), (0,)), ((), ())),
                preferred_element_type=jnp.float32,
            )
            uvT_ref[:, pl.ds(g * 128, 128)] = uvg

        pltpu.make_async_copy(itT_hbm, itT_ref, sem_i).wait()

    out_ref[...] = lax.dot_general(
        itT_ref[:, pl.ds(j * _BLOCK_N, _BLOCK_N)],
        uvT_ref[...],
        dimension_numbers=(((0,), (0,)), ((), ())),
        preferred_element_type=jnp.float32,
    )


_grid_spec = pltpu.PrefetchScalarGridSpec(
    num_scalar_prefetch=1,
    grid=(_NJ,),
    in_specs=[
        pl.BlockSpec(memory_space=pltpu.MemorySpace.HBM),
        pl.BlockSpec(memory_space=pltpu.MemorySpace.HBM),
        pl.BlockSpec((_G, 128), lambda j, idx: (0, 0)),
    ],
    out_specs=pl.BlockSpec((_BLOCK_N, BATCH), lambda j, idx: (j, 0)),
    scratch_shapes=[
        pltpu.VMEM((EMB, BATCH), jnp.float32),
        pltpu.VMEM((EMB, NUM_ITEMS), jnp.float32),
        pltpu.VMEM((EMB, 128 * BATCH), jnp.float32),
        pltpu.SemaphoreType.DMA((_G,)),
        pltpu.SemaphoreType.DMA,
    ],
)

_fused = pl.pallas_call(
    _body,
    grid_spec=_grid_spec,
    out_shape=jax.ShapeDtypeStruct((NUM_ITEMS, BATCH), jnp.float32),
    compiler_params=pltpu.CompilerParams(
        dimension_semantics=("arbitrary",),
    ),
)


def kernel(u_idx_tensor, u_emb, i_emb):
    idx = u_idx_tensor.astype(jnp.int32)
    idx2 = idx.reshape(_G, 128)
    return _fused(idx, u_emb.T, i_emb.T, idx2).T
